# baseline (speedup 1.0000x reference)
import functools

import jax
import jax.numpy as jnp
from jax import lax
from jax.experimental import pallas as pl
from jax.experimental.pallas import tpu as pltpu
from jax.experimental.pallas import tpu_sc as plsc

NC = 1
NS = 16
NW = NC * NS
LANES = 16
CHUNK = 128


def _make_sc_l1(n_rows: int, n_models: int, n_cols: int, k: int):
    rows_per_w = n_rows // NW
    n_chunks = k // CHUNK
    row_stride = n_cols
    mesh = plsc.VectorSubcoreMesh(core_axis_name="c", subcore_axis_name="s", num_cores=1)

    n_el = rows_per_w * k

    @functools.partial(
        pl.kernel,
        mesh=mesh,
        out_type=jax.ShapeDtypeStruct((NC, NS, LANES), jnp.float32),
        scratch_types=[
            pltpu.VMEM((k,), jnp.int32),
            pltpu.VMEM((n_el,), jnp.int32),
            pltpu.VMEM((n_el,), jnp.float32),
            pltpu.VMEM((n_el,), jnp.float32),
            pltpu.VMEM((LANES,), jnp.float32),
            pltpu.SemaphoreType.DMA,
        ],
    )
    def sc_l1(yv_hbm, bv_hbm, mask_hbm, out_hbm,
              mask_v, idx_v, p_v, t_v, acc_v, sem):
        c = lax.axis_index("c")
        s = lax.axis_index("s")
        wid = s * NC + c
        pltpu.sync_copy(mask_hbm, mask_v)
        for j in range(k // LANES):
            n16 = mask_v[pl.ds(j * LANES, LANES)]
            mask_v[pl.ds(j * LANES, LANES)] = (
                lax.shift_right_logical(n16, 7) * (n_models * 128)
                + (n16 & 127))
        for r in range(rows_per_w):
            row = wid * rows_per_w + r
            base = (lax.div(row, n_models) * (n_models * row_stride)
                    + lax.rem(row, n_models) * 128).astype(jnp.int32)
            for j in range(k // LANES):
                m16 = mask_v[pl.ds(j * LANES, LANES)]
                idx_v[pl.ds(r * k + j * LANES, LANES)] = m16 + base
        cp_p = pltpu.async_copy(yv_hbm.at[idx_v], p_v, sem)
        cp_t = pltpu.async_copy(bv_hbm.at[idx_v], t_v, sem)
        cp_p.wait()
        cp_t.wait()
        acc = jnp.zeros((LANES,), jnp.float32)
        for j in range(n_el // LANES):
            p16 = p_v[pl.ds(j * LANES, LANES)]
            t16 = t_v[pl.ds(j * LANES, LANES)]
            acc = acc + jnp.abs(p16 - t16)
        acc_v[...] = acc * jnp.float32(n_cols / k)
        pltpu.sync_copy(acc_v, out_hbm.at[c, s])

    return sc_l1


def kernel(y_hat, y_bar, mask):
    n_batch, n_models, n_cols = y_hat.shape
    k = mask.shape[0]
    n_t = n_cols // 128
    yv = (y_hat.reshape(n_batch, n_models, n_t, 128)
          .swapaxes(1, 2).reshape(-1))
    bv = (y_bar.reshape(n_batch, n_models, n_t, 128)
          .swapaxes(1, 2).reshape(-1))
    sc_l1 = _make_sc_l1(n_batch * n_models, n_models, n_cols, k)
    part = sc_l1(yv, bv, mask)
    return jnp.sum(part)


# speedup vs baseline: 1.1465x; 1.1465x over previous
import functools

import jax
import jax.numpy as jnp
from jax import lax
from jax.experimental import pallas as pl
from jax.experimental.pallas import tpu as pltpu
from jax.experimental.pallas import tpu_sc as plsc

NC = 2
NS = 16
NW = NC * NS
LANES = 16
CHUNK = 128


def _make_sc_l1(n_rows: int, n_models: int, n_cols: int, k: int):
    rows_per_w = n_rows // NW
    n_chunks = k // CHUNK
    row_stride = n_cols
    mesh = plsc.VectorSubcoreMesh(core_axis_name="c", subcore_axis_name="s")

    n_el = rows_per_w * k

    @functools.partial(
        pl.kernel,
        mesh=mesh,
        out_type=jax.ShapeDtypeStruct((NC, NS, LANES), jnp.float32),
        scratch_types=[
            pltpu.VMEM((k,), jnp.int32),
            pltpu.VMEM((n_el,), jnp.int32),
            pltpu.VMEM((n_el,), jnp.float32),
            pltpu.VMEM((n_el,), jnp.float32),
            pltpu.VMEM((LANES,), jnp.float32),
            pltpu.SemaphoreType.DMA,
        ],
    )
    def sc_l1(yv_hbm, bv_hbm, mask_hbm, out_hbm,
              mask_v, idx_v, p_v, t_v, acc_v, sem):
        c = lax.axis_index("c")
        s = lax.axis_index("s")
        wid = s * NC + c
        pltpu.sync_copy(mask_hbm, mask_v)
        for j in range(k // LANES):
            n16 = mask_v[pl.ds(j * LANES, LANES)]
            mask_v[pl.ds(j * LANES, LANES)] = (
                lax.shift_right_logical(n16, 7) * (n_models * 128)
                + (n16 & 127))
        for r in range(rows_per_w):
            row = wid * rows_per_w + r
            base = (lax.div(row, n_models) * (n_models * row_stride)
                    + lax.rem(row, n_models) * 128).astype(jnp.int32)
            for j in range(k // LANES):
                m16 = mask_v[pl.ds(j * LANES, LANES)]
                idx_v[pl.ds(r * k + j * LANES, LANES)] = m16 + base
        n_sl = 4
        sl_el = n_el // n_sl
        copies = []
        for t in range(n_sl):
            sl = pl.ds(t * sl_el, sl_el)
            copies.append((pltpu.async_copy(yv_hbm.at[idx_v.at[sl]],
                                            p_v.at[sl], sem),
                           pltpu.async_copy(bv_hbm.at[idx_v.at[sl]],
                                            t_v.at[sl], sem)))
        acc = jnp.zeros((LANES,), jnp.float32)
        for t, (cp_p, cp_t) in enumerate(copies):
            cp_p.wait()
            cp_t.wait()
            for j in range(sl_el // LANES):
                p16 = p_v[pl.ds(t * sl_el + j * LANES, LANES)]
                t16 = t_v[pl.ds(t * sl_el + j * LANES, LANES)]
                acc = acc + jnp.abs(p16 - t16)
        acc_v[...] = acc * jnp.float32(n_cols / k)
        pltpu.sync_copy(acc_v, out_hbm.at[c, s])

    return sc_l1


def kernel(y_hat, y_bar, mask):
    n_batch, n_models, n_cols = y_hat.shape
    k = mask.shape[0]
    n_t = n_cols // 128
    yv = (y_hat.reshape(n_batch, n_models, n_t, 128)
          .swapaxes(1, 2).reshape(-1))
    bv = (y_bar.reshape(n_batch, n_models, n_t, 128)
          .swapaxes(1, 2).reshape(-1))
    sc_l1 = _make_sc_l1(n_batch * n_models, n_models, n_cols, k)
    part = sc_l1(yv, bv, mask)
    return jnp.sum(part)


# speedup vs baseline: 1.1490x; 1.0022x over previous
import functools

import jax
import jax.numpy as jnp
from jax import lax
from jax.experimental import pallas as pl
from jax.experimental.pallas import tpu as pltpu
from jax.experimental.pallas import tpu_sc as plsc

NC = 2
NS = 16
NW = NC * NS
LANES = 16


def _make_sc_l1(n_rows: int, n_models: int, n_cols: int, k: int):
    rows_per_w = n_rows // NW
    row_stride = n_cols
    mesh = plsc.VectorSubcoreMesh(core_axis_name="c", subcore_axis_name="s")

    n_el = rows_per_w * k

    @functools.partial(
        pl.kernel,
        mesh=mesh,
        out_type=jax.ShapeDtypeStruct((NC, NS, LANES), jnp.float32),
        scratch_types=[
            pltpu.VMEM((k,), jnp.int32),
            pltpu.VMEM((n_el,), jnp.int32),
            pltpu.VMEM((n_el,), jnp.float32),
            pltpu.VMEM((n_el,), jnp.float32),
            pltpu.VMEM((LANES,), jnp.float32),
            pltpu.SemaphoreType.DMA,
        ],
    )
    def sc_l1(yv_hbm, bv_hbm, mask_hbm, out_hbm,
              mask_v, idx_v, p_v, t_v, acc_v, sem):
        c = lax.axis_index("c")
        s = lax.axis_index("s")
        wid = s * NC + c
        pltpu.sync_copy(mask_hbm, mask_v)
        for j in range(k // LANES):
            n16 = mask_v[pl.ds(j * LANES, LANES)]
            mask_v[pl.ds(j * LANES, LANES)] = (
                lax.shift_right_logical(n16, 7) * (n_models * 128)
                + (n16 & 127))
        for r in range(rows_per_w):
            row = wid * rows_per_w + r
            base = (lax.div(row, n_models) * (n_models * row_stride)
                    + lax.rem(row, n_models) * 128).astype(jnp.int32)
            for j in range(k // LANES):
                m16 = mask_v[pl.ds(j * LANES, LANES)]
                idx_v[pl.ds(r * k + j * LANES, LANES)] = m16 + base
        n_sl = 4
        sl_el = n_el // n_sl
        copies = []
        for t in range(n_sl):
            sl = pl.ds(t * sl_el, sl_el)
            copies.append((pltpu.async_copy(yv_hbm.at[idx_v.at[sl]],
                                            p_v.at[sl], sem),
                           pltpu.async_copy(bv_hbm.at[idx_v.at[sl]],
                                            t_v.at[sl], sem)))
        acc = jnp.zeros((LANES,), jnp.float32)
        for t, (cp_p, cp_t) in enumerate(copies):
            cp_p.wait()
            cp_t.wait()
            for j in range(sl_el // LANES):
                p16 = p_v[pl.ds(t * sl_el + j * LANES, LANES)]
                t16 = t_v[pl.ds(t * sl_el + j * LANES, LANES)]
                acc = acc + jnp.abs(p16 - t16)
        acc_v[...] = acc * jnp.float32(n_cols / k)
        pltpu.sync_copy(acc_v, out_hbm.at[c, s])

    return sc_l1


def kernel(y_hat, y_bar, mask):
    n_batch, n_models, n_cols = y_hat.shape
    k = mask.shape[0]
    n_t = n_cols // 128
    yv = (y_hat.reshape(n_batch, n_models, n_t, 128)
          .swapaxes(1, 2).reshape(-1))
    bv = (y_bar.reshape(n_batch, n_models, n_t, 128)
          .swapaxes(1, 2).reshape(-1))
    sc_l1 = _make_sc_l1(n_batch * n_models, n_models, n_cols, k)
    part = sc_l1(yv, bv, mask)
    return jnp.sum(part)
